# baseline (speedup 1.0000x reference)
import jax
import jax.numpy as jnp
from jax.experimental import pallas as pl
from jax.experimental.pallas import tpu as pltpu
from jax.experimental.pallas import tpu_sc as plsc

_WINDOW = 128
_TC1_BLOCK_V = 4096


def _cdiv(a, b):
    return (a + b - 1) // b


def kernel(token_ids, weight):
    b, s = token_ids.shape
    n = b * s
    v, d = weight.shape
    w_t = weight.T
    flat_ids = token_ids.reshape(1, n).astype(jnp.int32)

    def relayout_body(wt_ref, o_ref):
        o_ref[...] = jnp.transpose(wt_ref[...], (1, 0))

    table = pl.pallas_call(
        relayout_body,
        grid=(_cdiv(v, _TC1_BLOCK_V),),
        in_specs=[pl.BlockSpec((d, _TC1_BLOCK_V), lambda i: (0, i))],
        out_specs=pl.BlockSpec((_TC1_BLOCK_V, d), lambda i: (i, 0)),
        out_shape=jax.ShapeDtypeStruct((v, d), jnp.float32),
        compiler_params=pltpu.CompilerParams(
            dimension_semantics=("parallel",)),
    )(w_t)

    mesh = plsc.VectorSubcoreMesh(core_axis_name="c", subcore_axis_name="s")

    @pl.kernel(
        out_type=jax.ShapeDtypeStruct((n, d), jnp.float32),
        mesh=mesh,
        compiler_params=pltpu.CompilerParams(use_tc_tiling_on_sc=False),
    )
    def gather_kernel(w_hbm, i_hbm, o_hbm):
        def body(i_vmem, o_vmem):
            pltpu.sync_copy(w_hbm.at[i_vmem.at[0]], o_vmem)

        pltpu.emit_pipeline(
            body,
            grid=(n // _WINDOW,),
            in_specs=[pl.BlockSpec((1, _WINDOW), index_map=lambda i: (0, i))],
            out_specs=[pl.BlockSpec((_WINDOW, d), index_map=lambda i: (i, 0))],
            core_axis_name=("c", "s"),
            dimension_semantics=(pltpu.PARALLEL,),
        )(i_hbm, o_hbm)

    g = gather_kernel(table, flat_ids)
    return g.reshape(b, s, d)


# speedup vs baseline: 1.1462x; 1.1462x over previous
import jax
import jax.numpy as jnp
from jax.experimental import pallas as pl
from jax.experimental.pallas import tpu as pltpu
from jax.experimental.pallas import tpu_sc as plsc

_WINDOW = 128
_TC1_BLOCK_V = 4096


def _cdiv(a, b):
    return (a + b - 1) // b


def kernel(token_ids, weight):
    b, s = token_ids.shape
    n = b * s
    v, d = weight.shape
    flat_ids = token_ids.reshape(1, n).astype(jnp.int32)

    mesh = plsc.VectorSubcoreMesh(core_axis_name="c", subcore_axis_name="s")

    @pl.kernel(
        out_type=jax.ShapeDtypeStruct((n, d), jnp.float32),
        mesh=mesh,
        compiler_params=pltpu.CompilerParams(use_tc_tiling_on_sc=False),
    )
    def gather_kernel(w_hbm, i_hbm, o_hbm):
        def body(i_vmem, o_vmem):
            pltpu.sync_copy(w_hbm.at[i_vmem.at[0]], o_vmem)

        pltpu.emit_pipeline(
            body,
            grid=(n // _WINDOW,),
            in_specs=[pl.BlockSpec((1, _WINDOW), index_map=lambda i: (0, i))],
            out_specs=[pl.BlockSpec((_WINDOW, d), index_map=lambda i: (i, 0))],
            core_axis_name=("c", "s"),
            dimension_semantics=(pltpu.PARALLEL,),
        )(i_hbm, o_hbm)

    g = gather_kernel(weight, flat_ids)
    return g.reshape(b, s, d)


# speedup vs baseline: 1.1479x; 1.0015x over previous
import jax
import jax.numpy as jnp
from jax.experimental import pallas as pl
from jax.experimental.pallas import tpu as pltpu
from jax.experimental.pallas import tpu_sc as plsc

_WINDOW = 128
_STEP = 512


def _cdiv(a, b):
    return (a + b - 1) // b


def kernel(token_ids, weight):
    b, s = token_ids.shape
    n = b * s
    v, d = weight.shape
    flat_ids = token_ids.reshape(1, n).astype(jnp.int32)

    mesh = plsc.VectorSubcoreMesh(core_axis_name="c", subcore_axis_name="s")

    @pl.kernel(
        out_type=jax.ShapeDtypeStruct((n, d), jnp.float32),
        mesh=mesh,
        compiler_params=pltpu.CompilerParams(use_tc_tiling_on_sc=False),
    )
    def gather_kernel(w_hbm, i_hbm, o_hbm):
        def body(i_vmem, o_vmem):
            for k in range(_STEP // _WINDOW):
                pltpu.sync_copy(
                    w_hbm.at[i_vmem.at[0, pl.ds(k * _WINDOW, _WINDOW)]],
                    o_vmem.at[pl.ds(k * _WINDOW, _WINDOW), :],
                )

        pltpu.emit_pipeline(
            body,
            grid=(n // _STEP,),
            in_specs=[pl.BlockSpec((1, _STEP), index_map=lambda i: (0, i))],
            out_specs=[pl.BlockSpec((_STEP, d), index_map=lambda i: (i, 0))],
            core_axis_name=("c", "s"),
            dimension_semantics=(pltpu.PARALLEL,),
        )(i_hbm, o_hbm)

    g = gather_kernel(weight, flat_ids)
    return g.reshape(b, s, d)
